# Initial kernel scaffold; baseline (speedup 1.0000x reference)
import functools

import jax
import jax.numpy as jnp
from jax import lax
from jax.experimental import pallas as pl
from jax.experimental.pallas import tpu as pltpu
from jax.experimental.pallas import tpu_sc as plsc

NUM_EMB = 1000000
NUM_BUCKETS = 100000
DIM = 64
BATCH = 4096
NFIELDS = 26
FLEN = 20

NBAGS = BATCH * NFIELDS
NW = 32
BAGS_PER_W = NBAGS // NW
C_BAGS = 32
C_ELEMS = C_BAGS * FLEN
N_CHUNKS = BAGS_PER_W // C_BAGS
N_IDX_ROWS = C_ELEMS // 128

_mesh = plsc.VectorSubcoreMesh(core_axis_name="c", subcore_axis_name="s")


@functools.partial(
    pl.kernel,
    out_type=jax.ShapeDtypeStruct((NBAGS, DIM), jnp.float32),
    mesh=_mesh,
    scratch_types=[
        pltpu.VMEM(((NUM_EMB // NUM_BUCKETS + 1) * DIM,), jnp.float32),
        pltpu.VMEM((C_ELEMS,), jnp.int32),
        pltpu.VMEM((N_IDX_ROWS, 128), jnp.int32),
        pltpu.VMEM((C_ELEMS,), jnp.int32),
        pltpu.VMEM((C_ELEMS, DIM), jnp.float32),
        pltpu.VMEM((C_BAGS, DIM), jnp.float32),
        pltpu.SemaphoreType.DMA,
    ],
)
def _embedding_bags_sc(x_hbm, w1_hbm, w2_hbm, out_hbm,
                       w2_v, xv, ridx, q64v, rows, outv, sem):
    wid = lax.axis_index("s") * 2 + lax.axis_index("c")

    pltpu.sync_copy(w2_hbm, w2_v)

    dg_iota = [lax.iota(jnp.int32, (16,)) + dg * 16 for dg in range(4)]

    @pl.loop(0, N_CHUNKS)
    def _chunk(c):
        bag0 = wid * BAGS_PER_W + c * C_BAGS
        e0 = bag0 * FLEN
        pltpu.sync_copy(x_hbm.at[pl.ds(e0, C_ELEMS)], xv)

        for j in range(C_ELEMS // 16):
            xi = xv[pl.ds(j * 16, 16)]
            q0 = (xi.astype(jnp.float32) * (1.0 / NUM_BUCKETS)).astype(jnp.int32)
            r0 = xi - q0 * NUM_BUCKETS
            neg = r0 < 0
            q0 = jnp.where(neg, q0 - 1, q0)
            r0 = jnp.where(neg, r0 + NUM_BUCKETS, r0)
            ovf = r0 >= NUM_BUCKETS
            q0 = jnp.where(ovf, q0 + 1, q0)
            r0 = jnp.where(ovf, r0 - NUM_BUCKETS, r0)
            live = xi != 0
            r = jnp.where(live, r0 + 1, 0)
            q64 = jnp.where(live, (q0 + 1) * DIM, 0)
            ridx[j // 8, pl.ds((j % 8) * 16, 16)] = r
            q64v[pl.ds(j * 16, 16)] = q64

        descs = [
            pltpu.async_copy(w1_hbm.at[ridx.at[k]],
                             rows.at[pl.ds(k * 128, 128)], sem)
            for k in range(N_IDX_ROWS)
        ]
        for d in descs:
            d.wait()

        @pl.loop(0, C_BAGS)
        def _bag(bb):
            e_base = bb * FLEN
            accs = [jnp.zeros((16,), jnp.float32) for _ in range(4)]
            for i in range(FLEN):
                e = e_base + i
                qb = plsc.load_gather(q64v, [jnp.full((16,), e, jnp.int32)])
                for dg in range(4):
                    w2 = plsc.load_gather(w2_v, [qb + dg_iota[dg]])
                    w1 = rows[e, pl.ds(dg * 16, 16)]
                    accs[dg] = accs[dg] + w1 * w2
            for dg in range(4):
                outv[bb, pl.ds(dg * 16, 16)] = accs[dg]

        pltpu.sync_copy(outv, out_hbm.at[pl.ds(bag0, C_BAGS)])


def kernel(x, W1, W2):
    out = _embedding_bags_sc(x.reshape(-1).astype(jnp.int32), W1,
                             W2.reshape(-1))
    return out.reshape(BATCH, NFIELDS, DIM)


# speedup vs baseline: 21.7236x; 21.7236x over previous
import functools

import jax
import jax.numpy as jnp
from jax import lax
from jax.experimental import pallas as pl
from jax.experimental.pallas import tpu as pltpu
from jax.experimental.pallas import tpu_sc as plsc

NUM_EMB = 1000000
NUM_BUCKETS = 100000
DIM = 64
BATCH = 4096
NFIELDS = 26
FLEN = 20

NBAGS = BATCH * NFIELDS
NW = 32
BAGS_PER_W = NBAGS // NW
C_BAGS = 32
C_ELEMS = C_BAGS * FLEN
N_CHUNKS = BAGS_PER_W // C_BAGS
N_IDX_ROWS = C_ELEMS // 128

_mesh = plsc.VectorSubcoreMesh(core_axis_name="c", subcore_axis_name="s")


@functools.partial(
    pl.kernel,
    out_type=jax.ShapeDtypeStruct((NBAGS, DIM), jnp.float32),
    mesh=_mesh,
    scratch_types=[
        pltpu.VMEM(((NUM_EMB // NUM_BUCKETS + 1) * DIM,), jnp.float32),
        pltpu.VMEM((C_ELEMS,), jnp.int32),
        pltpu.VMEM((N_IDX_ROWS, 128), jnp.int32),
        pltpu.VMEM((C_ELEMS,), jnp.int32),
        pltpu.VMEM((C_ELEMS, DIM), jnp.float32),
        pltpu.VMEM((C_BAGS, DIM), jnp.float32),
        pltpu.SemaphoreType.DMA,
    ],
    compiler_params=pltpu.CompilerParams(needs_layout_passes=False,
                                         use_tc_tiling_on_sc=False),
)
def _embedding_bags_sc(x_hbm, w1_hbm, w2_hbm, out_hbm,
                       w2_v, xv, ridx, q64v, rows, outv, sem):
    wid = lax.axis_index("s") * 2 + lax.axis_index("c")

    pltpu.sync_copy(w2_hbm, w2_v)

    dg_iota = [lax.iota(jnp.int32, 16) + dg * 16 for dg in range(4)]

    @pl.loop(0, N_CHUNKS)
    def _chunk(c):
        bag0 = wid * BAGS_PER_W + c * C_BAGS
        e0 = bag0 * FLEN
        pltpu.sync_copy(x_hbm.at[pl.ds(e0, C_ELEMS)], xv)

        for j in range(C_ELEMS // 16):
            xi = xv[pl.ds(j * 16, 16)]
            q0 = (xi.astype(jnp.float32) * (1.0 / NUM_BUCKETS)).astype(jnp.int32)
            r0 = xi - q0 * NUM_BUCKETS
            neg = r0 < 0
            q0 = jnp.where(neg, q0 - 1, q0)
            r0 = jnp.where(neg, r0 + NUM_BUCKETS, r0)
            ovf = r0 >= NUM_BUCKETS
            q0 = jnp.where(ovf, q0 + 1, q0)
            r0 = jnp.where(ovf, r0 - NUM_BUCKETS, r0)
            live = xi != 0
            r = jnp.where(live, r0 + 1, 0)
            q64 = jnp.where(live, (q0 + 1) * DIM, 0)
            ridx[j // 8, pl.ds((j % 8) * 16, 16)] = r
            q64v[pl.ds(j * 16, 16)] = q64

        descs = [
            pltpu.async_copy(w1_hbm.at[ridx.at[k]],
                             rows.at[pl.ds(k * 128, 128)], sem)
            for k in range(N_IDX_ROWS)
        ]
        for d in descs:
            d.wait()

        @pl.loop(0, C_BAGS)
        def _bag(bb):
            e_base = bb * FLEN
            accs = [jnp.zeros((16,), jnp.float32) for _ in range(4)]
            for i in range(FLEN):
                e = e_base + i
                qb = plsc.load_gather(q64v, [jnp.full((16,), e, jnp.int32)])
                for dg in range(4):
                    w2 = plsc.load_gather(w2_v, [qb + dg_iota[dg]])
                    w1 = rows[e, pl.ds(dg * 16, 16)]
                    accs[dg] = accs[dg] + w1 * w2
            for dg in range(4):
                outv[bb, pl.ds(dg * 16, 16)] = accs[dg]

        pltpu.sync_copy(outv, out_hbm.at[pl.ds(bag0, C_BAGS)])


def kernel(x, W1, W2):
    out = _embedding_bags_sc(x.reshape(-1).astype(jnp.int32), W1,
                             W2.reshape(-1))
    return out.reshape(BATCH, NFIELDS, DIM)


# speedup vs baseline: 25.7970x; 1.1875x over previous
import functools

import jax
import jax.numpy as jnp
from jax import lax
from jax.experimental import pallas as pl
from jax.experimental.pallas import tpu as pltpu
from jax.experimental.pallas import tpu_sc as plsc

NUM_EMB = 1000000
NUM_BUCKETS = 100000
DIM = 64
BATCH = 4096
NFIELDS = 26
FLEN = 20

NBAGS = BATCH * NFIELDS
NW = 32
BAGS_PER_W = NBAGS // NW
C_BAGS = 32
C_ELEMS = C_BAGS * FLEN
N_CHUNKS = BAGS_PER_W // C_BAGS
N_IDX_ROWS = C_ELEMS // 128

_mesh = plsc.VectorSubcoreMesh(core_axis_name="c", subcore_axis_name="s")


@functools.partial(
    pl.kernel,
    out_type=jax.ShapeDtypeStruct((NBAGS, DIM), jnp.float32),
    mesh=_mesh,
    scratch_types=[
        pltpu.VMEM(((NUM_EMB // NUM_BUCKETS + 1) * DIM,), jnp.float32),
        [pltpu.VMEM((C_ELEMS,), jnp.int32)] * 2,
        [pltpu.VMEM((N_IDX_ROWS, 128), jnp.int32)] * 2,
        [pltpu.VMEM((C_ELEMS,), jnp.int32)] * 2,
        [pltpu.VMEM((C_ELEMS, DIM), jnp.float32)] * 2,
        pltpu.VMEM((C_BAGS, DIM), jnp.float32),
        [pltpu.SemaphoreType.DMA] * 2,
    ],
    compiler_params=pltpu.CompilerParams(needs_layout_passes=False,
                                         use_tc_tiling_on_sc=False),
)
def _embedding_bags_sc(x_hbm, w1_hbm, w2_hbm, out_hbm,
                       w2_v, xvs, ridxs, q64vs, rowss, outv, sems):
    wid = lax.axis_index("s") * 2 + lax.axis_index("c")

    pltpu.sync_copy(w2_hbm, w2_v)

    dg_iota = [lax.iota(jnp.int32, 16) + dg * 16 for dg in range(4)]

    def launch(c, p):
        e0 = (wid * BAGS_PER_W + c * C_BAGS) * FLEN
        pltpu.sync_copy(x_hbm.at[pl.ds(e0, C_ELEMS)], xvs[p])
        for j in range(C_ELEMS // 16):
            xi = xvs[p][pl.ds(j * 16, 16)]
            q0 = (xi.astype(jnp.float32) * (1.0 / NUM_BUCKETS)).astype(jnp.int32)
            r0 = xi - q0 * NUM_BUCKETS
            neg = r0 < 0
            q0 = jnp.where(neg, q0 - 1, q0)
            r0 = jnp.where(neg, r0 + NUM_BUCKETS, r0)
            ovf = r0 >= NUM_BUCKETS
            q0 = jnp.where(ovf, q0 + 1, q0)
            r0 = jnp.where(ovf, r0 - NUM_BUCKETS, r0)
            live = xi != 0
            ridxs[p][j // 8, pl.ds((j % 8) * 16, 16)] = jnp.where(live, r0 + 1, 0)
            q64vs[p][pl.ds(j * 16, 16)] = jnp.where(live, (q0 + 1) * DIM, 0)
        for k in range(N_IDX_ROWS):
            pltpu.async_copy(w1_hbm.at[ridxs[p].at[k]],
                             rowss[p].at[pl.ds(k * 128, 128)], sems[p])

    def drain(p):
        for k in range(N_IDX_ROWS):
            pltpu.make_async_copy(w1_hbm.at[ridxs[p].at[k]],
                                  rowss[p].at[pl.ds(k * 128, 128)],
                                  sems[p]).wait()

    def combine(c, p):
        rows, q64v = rowss[p], q64vs[p]

        @pl.loop(0, C_BAGS)
        def _bag(bb):
            e_base = bb * FLEN
            accs = [jnp.zeros((16,), jnp.float32) for _ in range(4)]
            for i in range(FLEN):
                e = e_base + i
                qb = plsc.load_gather(q64v, [jnp.full((16,), e, jnp.int32)])
                for dg in range(4):
                    w2 = plsc.load_gather(w2_v, [qb + dg_iota[dg]])
                    w1 = rows[e, pl.ds(dg * 16, 16)]
                    accs[dg] = accs[dg] + w1 * w2
            for dg in range(4):
                outv[bb, pl.ds(dg * 16, 16)] = accs[dg]

        pltpu.sync_copy(outv, out_hbm.at[pl.ds(wid * BAGS_PER_W + c * C_BAGS,
                                               C_BAGS)])

    launch(0, 0)

    @pl.loop(0, N_CHUNKS - 2, step=2)
    def _pair(c):
        launch(c + 1, 1)
        drain(0)
        combine(c, 0)
        launch(c + 2, 0)
        drain(1)
        combine(c + 1, 1)

    launch(N_CHUNKS - 1, 1)
    drain(0)
    combine(N_CHUNKS - 2, 0)
    drain(1)
    combine(N_CHUNKS - 1, 1)


def kernel(x, W1, W2):
    out = _embedding_bags_sc(x.reshape(-1).astype(jnp.int32), W1,
                             W2.reshape(-1))
    return out.reshape(BATCH, NFIELDS, DIM)
